# baseline (speedup 1.0000x reference)
import jax
import jax.numpy as jnp
from jax import lax
from jax.experimental import pallas as pl
from jax.experimental.pallas import tpu as pltpu
from jax.experimental.pallas import tpu_sc as plsc

NUM_ENTITIES = 1000000
NUM_RELATIONS = 1000
EMBED_DIM = 64
PK = 128
BATCH = 16384

_info = plsc.get_sparse_core_info()
NC, NS, L = _info.num_cores, _info.num_subcores, _info.num_lanes
NW = NC * NS

BLK = 128
N_FULL = NUM_ENTITIES // BLK
TAIL0 = N_FULL * BLK
TAIL = NUM_ENTITIES - TAIL0
SLOTS = (N_FULL + NW - 1) // NW

RPW = BATCH // NW
CHUNK = 128
N_CHUNKS = RPW // CHUNK
D_VECS = EMBED_DIM // L


def _tr_issue_in(pr_t, pi_t, col0, bufset, sem):
    sr_v, si_v, _ = bufset
    pltpu.async_copy(pr_t.at[:, pl.ds(col0, BLK)], sr_v, sem)
    pltpu.async_copy(pi_t.at[:, pl.ds(col0, BLK)], si_v, sem)


def _tr_drain_in(pr_t, bufset, sem):
    sr_v, si_v, _ = bufset
    pltpu.make_async_copy(pr_t.at[:, pl.ds(0, BLK)], sr_v, sem).wait()
    pltpu.make_async_copy(pr_t.at[:, pl.ds(0, BLK)], si_v, sem).wait()


def _tr_compute(bufset, n_rows):
    sr_v, si_v, t_v = bufset
    iotas = [lax.iota(jnp.int32, L) + cb * L for cb in range(D_VECS)]

    def row_body(e, carry):
        ev = jnp.full((L,), e, jnp.int32)
        for cb in range(D_VECS):
            t_v[e, pl.ds(cb * L, L)] = plsc.load_gather(sr_v, [iotas[cb], ev])
            t_v[e, pl.ds(EMBED_DIM + cb * L, L)] = plsc.load_gather(
                si_v, [iotas[cb], ev])
        return carry

    lax.fori_loop(0, n_rows, row_body, 0)


def _transpose_body(pr_t, pi_t, packed,
                    sr0, si0, t0, sr1, si1, t1,
                    in_sem0, in_sem1, out_sem):
    wid = lax.axis_index("s") * NC + lax.axis_index("c")
    bufs = ((sr0, si0, t0), (sr1, si1, t1))
    in_sems = (in_sem0, in_sem1)

    def blk_of(slot):
        return wid + NW * slot

    @pl.when(blk_of(0) < N_FULL)
    def _():
        _tr_issue_in(pr_t, pi_t, blk_of(0) * BLK, bufs[0], in_sems[0])

    def slot_work(slot, par):
        blk = blk_of(slot)

        @pl.when(blk_of(slot + 1) < N_FULL)
        def _():
            _tr_issue_in(pr_t, pi_t, blk_of(slot + 1) * BLK,
                         bufs[1 - par], in_sems[1 - par])

        @pl.when((slot >= 2) & (blk_of(slot - 2) < N_FULL))
        def _():
            pltpu.make_async_copy(packed.at[pl.ds(0, BLK)],
                                  bufs[par][2], out_sem).wait()

        @pl.when(blk < N_FULL)
        def _():
            _tr_drain_in(pr_t, bufs[par], in_sems[par])
            _tr_compute(bufs[par], BLK)
            pltpu.async_copy(bufs[par][2],
                             packed.at[pl.ds(blk * BLK, BLK)], out_sem)

    def pair_body(i, carry):
        slot_work(2 * i, 0)
        slot_work(2 * i + 1, 1)
        return carry

    n_pairs = (SLOTS + 2) // 2
    lax.fori_loop(0, n_pairs, pair_body, 0)

    for last in (2 * n_pairs - 2, 2 * n_pairs - 1):
        @pl.when(blk_of(last) < N_FULL)
        def _():
            pltpu.make_async_copy(packed.at[pl.ds(0, BLK)],
                                  bufs[last % 2][2], out_sem).wait()

    @pl.when(wid == 0)
    def _():
        sr_v, si_v, t_v = bufs[0]
        for j in range(EMBED_DIM):
            pltpu.async_copy(pr_t.at[j, pl.ds(TAIL0, TAIL)],
                             sr_v.at[j, pl.ds(0, TAIL)], in_sems[0])
            pltpu.async_copy(pi_t.at[j, pl.ds(TAIL0, TAIL)],
                             si_v.at[j, pl.ds(0, TAIL)], in_sems[0])
        for j in range(EMBED_DIM):
            pltpu.make_async_copy(pr_t.at[0, pl.ds(0, TAIL)],
                                  sr_v.at[j, pl.ds(0, TAIL)],
                                  in_sems[0]).wait()
            pltpu.make_async_copy(pr_t.at[0, pl.ds(0, TAIL)],
                                  si_v.at[j, pl.ds(0, TAIL)],
                                  in_sems[0]).wait()
        _tr_compute(bufs[0], TAIL)
        pltpu.sync_copy(t_v.at[pl.ds(0, TAIL)],
                        packed.at[pl.ds(TAIL0, TAIL)])


def _g_issue(entpk, relpk, eidx_v, ridx_v, ci, bufset, sem):
    e_v, r_v = bufset
    sl = pl.ds(ci * CHUNK, CHUNK)
    pltpu.async_copy(entpk.at[eidx_v.at[sl]], e_v, sem)
    pltpu.async_copy(relpk.at[ridx_v.at[sl]], r_v, sem)


def _g_drain(entpk, relpk, bufset, sem):
    e_v, r_v = bufset
    pltpu.make_async_copy(entpk.at[pl.ds(0, CHUNK)], e_v, sem).wait()
    pltpu.make_async_copy(relpk.at[pl.ds(0, CHUNK)], r_v, sem).wait()


def _g_compute(bufset, or_v, oi_v):
    e_v, r_v = bufset

    def row_body(row, carry):
        for cb in range(D_VECS):
            sl = pl.ds(cb * L, L)
            sli = pl.ds(EMBED_DIM + cb * L, L)
            a = e_v[row, sl]
            b = e_v[row, sli]
            cc = r_v[row, sl]
            d = r_v[row, sli]
            or_v[row, sl] = a * cc - b * d
            oi_v[row, sl] = a * d + b * cc
        return carry

    lax.fori_loop(0, CHUNK, row_body, 0)


def _gather_body(e1_hbm, r_hbm, entpk, relpk, out_r, out_i,
                 eidx_v, ridx_v, e0, r0, e1b, r1b, or_v, oi_v, sem0, sem1):
    wid = lax.axis_index("s") * NC + lax.axis_index("c")
    base = wid * RPW
    pltpu.sync_copy(e1_hbm.at[pl.ds(base, RPW)], eidx_v)
    pltpu.sync_copy(r_hbm.at[pl.ds(base, RPW)], ridx_v)

    bufs = ((e0, r0), (e1b, r1b))
    sems = (sem0, sem1)
    _g_issue(entpk, relpk, eidx_v, ridx_v, 0, bufs[0], sems[0])
    for ci in range(N_CHUNKS):
        par = ci % 2
        if ci + 1 < N_CHUNKS:
            _g_issue(entpk, relpk, eidx_v, ridx_v, ci + 1,
                     bufs[1 - par], sems[1 - par])
        _g_drain(entpk, relpk, bufs[par], sems[par])
        _g_compute(bufs[par], or_v, oi_v)
        off = base + ci * CHUNK
        pltpu.sync_copy(or_v, out_r.at[pl.ds(off, CHUNK)])
        pltpu.sync_copy(oi_v, out_i.at[pl.ds(off, CHUNK)])


@jax.jit
def kernel(e1, r, ent_real, ent_img, rel_real, rel_img):
    mesh = plsc.VectorSubcoreMesh(core_axis_name="c", subcore_axis_name="s")
    params = pltpu.CompilerParams(
        use_tc_tiling_on_sc=True, needs_layout_passes=False,
        disable_bounds_checks=True)

    pr_t = ent_real.T
    pi_t = ent_img.T
    stage = pltpu.VMEM((EMBED_DIM, BLK), jnp.float32)
    tbuf = pltpu.VMEM((BLK, PK), jnp.float32)
    transpose_fn = pl.kernel(
        _transpose_body,
        out_type=jax.ShapeDtypeStruct((NUM_ENTITIES, PK), jnp.float32),
        mesh=mesh,
        scratch_types=[
            stage, stage, tbuf, stage, stage, tbuf,
            pltpu.SemaphoreType.DMA,
            pltpu.SemaphoreType.DMA,
            pltpu.SemaphoreType.DMA,
        ],
        compiler_params=params,
    )
    entpk = transpose_fn(pr_t, pi_t)
    relpk = jnp.concatenate([rel_real, rel_img], axis=1)

    out_shape = jax.ShapeDtypeStruct((BATCH, EMBED_DIM), jnp.float32)
    gbuf = pltpu.VMEM((CHUNK, PK), jnp.float32)
    gather_fn = pl.kernel(
        _gather_body,
        out_type=(out_shape, out_shape),
        mesh=mesh,
        scratch_types=[
            pltpu.VMEM((RPW,), jnp.int32),
            pltpu.VMEM((RPW,), jnp.int32),
            gbuf, gbuf, gbuf, gbuf,
            pltpu.VMEM((CHUNK, EMBED_DIM), jnp.float32),
            pltpu.VMEM((CHUNK, EMBED_DIM), jnp.float32),
            pltpu.SemaphoreType.DMA,
            pltpu.SemaphoreType.DMA,
        ],
        compiler_params=params,
    )
    return gather_fn(e1, r, entpk, relpk)


# speedup vs baseline: 5.9079x; 5.9079x over previous
import jax
import jax.numpy as jnp
from jax import lax
from jax.experimental import pallas as pl
from jax.experimental.pallas import tpu as pltpu
from jax.experimental.pallas import tpu_sc as plsc

NUM_ENTITIES = 1000000
NUM_RELATIONS = 1000
EMBED_DIM = 64
PK = 128
BATCH = 16384

_info = plsc.get_sparse_core_info()
NC, NS, L = _info.num_cores, _info.num_subcores, _info.num_lanes
NW = NC * NS
RPW = BATCH // NW
G = 16
CHUNK = 128
GROUPS_PER_CHUNK = CHUNK // G
N_CHUNKS = RPW // CHUNK
D_VECS = EMBED_DIM // L


def _body(e1_hbm, r_hbm, er3, ei3, relpk, out_r, out_i,
          eidx_v, ridx_v, a_v, b_v, rel_v, or_v, oi_v, sem, rsem):
    wid = lax.axis_index("s") * NC + lax.axis_index("c")
    base = wid * RPW
    pltpu.sync_copy(e1_hbm.at[pl.ds(base, RPW)], eidx_v)
    pltpu.sync_copy(r_hbm.at[pl.ds(base, RPW)], ridx_v)

    def chunk_body(ci, carry):
        pltpu.async_copy(relpk.at[ridx_v.at[pl.ds(ci * CHUNK, CHUNK)]],
                         rel_v, rsem)
        pltpu.make_async_copy(relpk.at[pl.ds(0, CHUNK)], rel_v, rsem).wait()

        def grp_body(g, gcarry):
            grow = ci * GROUPS_PER_CHUNK + g
            e_vec = eidx_v[pl.ds(grow * G, G)]
            for j in range(G):
                pe = e_vec[j] >> 3
                se = e_vec[j] & 7
                pltpu.async_copy(er3.at[pe, se], a_v.at[j], sem)
                pltpu.async_copy(ei3.at[pe, se], b_v.at[j], sem)
            for j in range(G):
                pltpu.make_async_copy(er3.at[0, 0], a_v.at[j], sem).wait()
                pltpu.make_async_copy(er3.at[0, 0], b_v.at[j], sem).wait()
            row0 = g * G
            for j in range(G):
                for cb in range(D_VECS):
                    sl = pl.ds(cb * L, L)
                    sli = pl.ds(EMBED_DIM + cb * L, L)
                    a = a_v[j, sl]
                    b = b_v[j, sl]
                    cc = rel_v[row0 + j, sl]
                    d = rel_v[row0 + j, sli]
                    or_v[row0 + j, sl] = a * cc - b * d
                    oi_v[row0 + j, sl] = a * d + b * cc
            return gcarry

        lax.fori_loop(0, GROUPS_PER_CHUNK, grp_body, 0)
        off = base + ci * CHUNK
        pltpu.sync_copy(or_v, out_r.at[pl.ds(off, CHUNK)])
        pltpu.sync_copy(oi_v, out_i.at[pl.ds(off, CHUNK)])
        return carry

    lax.fori_loop(0, N_CHUNKS, chunk_body, 0)


@jax.jit
def kernel(e1, r, ent_real, ent_img, rel_real, rel_img):
    er3 = ent_real.reshape(NUM_ENTITIES // 8, 8, EMBED_DIM)
    ei3 = ent_img.reshape(NUM_ENTITIES // 8, 8, EMBED_DIM)
    relpk = jnp.concatenate([rel_real, rel_img], axis=1)
    mesh = plsc.VectorSubcoreMesh(core_axis_name="c", subcore_axis_name="s")
    out_shape = jax.ShapeDtypeStruct((BATCH, EMBED_DIM), jnp.float32)
    fn = pl.kernel(
        _body,
        out_type=(out_shape, out_shape),
        mesh=mesh,
        scratch_types=[
            pltpu.VMEM((RPW,), jnp.int32),
            pltpu.VMEM((RPW,), jnp.int32),
            pltpu.VMEM((G, EMBED_DIM), jnp.float32),
            pltpu.VMEM((G, EMBED_DIM), jnp.float32),
            pltpu.VMEM((CHUNK, PK), jnp.float32),
            pltpu.VMEM((CHUNK, EMBED_DIM), jnp.float32),
            pltpu.VMEM((CHUNK, EMBED_DIM), jnp.float32),
            pltpu.SemaphoreType.DMA,
            pltpu.SemaphoreType.DMA,
        ],
        compiler_params=pltpu.CompilerParams(
            use_tc_tiling_on_sc=True, needs_layout_passes=False),
    )
    return fn(e1, r, er3, ei3, relpk)
